# baseline (speedup 1.0000x reference)
import functools

import jax
import jax.numpy as jnp
from jax import lax
from jax.experimental import pallas as pl
from jax.experimental.pallas import tpu as pltpu
from jax.experimental.pallas import tpu_sc as plsc

QUES = 2048
SEQH = 2048
T = 8192
K = 32

GA = 8
AROWS = T // GA
GW = 16
WROWS = (3 * SEQH) // GW

NSUB = 16
RPS = T // NSUB
NEG = -3.0e38


def _alpha_body(q_ref, ques_ref, alpha_ref):
    alpha_ref[...] = jnp.dot(ques_ref[...], q_ref[...],
                             preferred_element_type=jnp.float32)


def _matvec_body(sel_ref, q_ref, h_ref, wih_ref, whh_ref, gi_ref, gh_ref):
    gi_ref[...] = jnp.dot(wih_ref[...], q_ref[...],
                          preferred_element_type=jnp.float32)
    gh_ref[...] = jnp.dot(whh_ref[...], h_ref[...],
                          preferred_element_type=jnp.float32)


def _sc_topk_body(alpha, idx_out, val_out,
                  av, lvv, liv, cvv, civ, sv_sh, si_sh):
    cid = lax.axis_index("c")
    sid = lax.axis_index("s")
    lane = lax.broadcasted_iota(jnp.int32, (16,), 0)
    m0 = lane == 0
    zf = jnp.zeros((16,), jnp.float32)
    zi = jnp.zeros((16,), jnp.int32)

    @pl.when(cid == 0)
    def _():
        base = pl.multiple_of(sid * RPS, RPS)
        pltpu.sync_copy(alpha.at[pl.ds(base, RPS)], av)

        def fold(t, carry):
            vb, ib = carry
            v = av[pl.ds(t * 16, 16)]
            gidx = base + t * 16 + lane
            better = v > vb
            return jnp.where(better, v, vb), jnp.where(better, gidx, ib)

        def ex(p, _):
            vb, ib = lax.fori_loop(1, RPS // 16, fold,
                                   (av[pl.ds(0, 16)], base + lane))
            mval = jnp.max(vb)
            midx = jnp.max(jnp.where(vb == mval, ib, jnp.int32(-1)))
            pv = zi + p
            plsc.store_scatter(lvv, [pv], zf + mval, mask=m0)
            plsc.store_scatter(liv, [pv], zi + midx, mask=m0)
            plsc.store_scatter(av, [zi + (midx - base)], zf + NEG, mask=m0)
            return 0

        lax.fori_loop(0, K, ex, 0)
        pltpu.sync_copy(lvv, sv_sh.at[pl.ds(sid * K, K)])
        pltpu.sync_copy(liv, si_sh.at[pl.ds(sid * K, K)])
        plsc.subcore_barrier()

        @pl.when(sid == 0)
        def _():
            pltpu.sync_copy(sv_sh, cvv)
            pltpu.sync_copy(si_sh, civ)

            def gfold(t, carry):
                vb, ib = carry
                v = cvv[pl.ds(t * 16, 16)]
                slot = t * 16 + lane
                better = v > vb
                return jnp.where(better, v, vb), jnp.where(better, slot, ib)

            def gex(p, _):
                vb, ib = lax.fori_loop(1, NSUB * K // 16, gfold,
                                       (cvv[pl.ds(0, 16)], lane))
                mval = jnp.max(vb)
                mslot = jnp.max(jnp.where(vb == mval, ib, jnp.int32(-1)))
                slotv = zi + mslot
                orig = plsc.load_gather(civ, [slotv])
                pv = zi + p
                plsc.store_scatter(lvv, [pv], zf + mval, mask=m0)
                plsc.store_scatter(liv, [pv], orig, mask=m0)
                plsc.store_scatter(cvv, [slotv], zf + NEG, mask=m0)
                return 0

            lax.fori_loop(0, K, gex, 0)
            pltpu.sync_copy(liv, idx_out)
            pltpu.sync_copy(lvv, val_out)


def _sc_topk(alpha):
    f32 = jnp.float32
    i32 = jnp.int32
    mesh = plsc.VectorSubcoreMesh(core_axis_name="c", subcore_axis_name="s")
    return pl.kernel(
        _sc_topk_body,
        mesh=mesh,
        compiler_params=pltpu.CompilerParams(needs_layout_passes=False,
                                             use_tc_tiling_on_sc=False),
        out_type=[
            jax.ShapeDtypeStruct((K,), i32),
            jax.ShapeDtypeStruct((K,), f32),
        ],
        scratch_types=[
            pltpu.VMEM((RPS,), f32),
            pltpu.VMEM((K,), f32),
            pltpu.VMEM((K,), i32),
            pltpu.VMEM((NSUB * K,), f32),
            pltpu.VMEM((NSUB * K,), i32),
            pltpu.VMEM_SHARED((NSUB * K,), f32),
            pltpu.VMEM_SHARED((NSUB * K,), i32),
        ],
    )(alpha)


def _final_body(idx_ref, val_ref, row_ref, q_ref, ws_ref, bs_ref,
                gi_ref, gh_ref, h_ref, bih_ref, bhh_ref,
                pred_ref, hnew_ref, acc_ref, w_ref):
    i = pl.program_id(0)

    @pl.when(i == 0)
    def _():
        v = val_ref[...]
        e = jnp.exp(v - jnp.max(v))
        w_ref[...] = e / jnp.sum(e)
        acc_ref[...] = jnp.zeros_like(acc_ref)

    kiota = lax.broadcasted_iota(jnp.int32, (1, K), 1)
    wi = jnp.sum(jnp.where(kiota == i, w_ref[...], 0.0))
    acc_ref[...] += wi * row_ref[0]

    @pl.when(i == K - 1)
    def _():
        ws = ws_ref[...]
        pred = (jnp.sum(ws[0:1] * q_ref[...])
                + jnp.sum(ws[1:2] * acc_ref[...]) + bs_ref[0, 0])
        pred_ref[...] = pred[None, None]
        gi = gi_ref[...] + bih_ref[...]
        gh = gh_ref[...] + bhh_ref[...]
        h = h_ref[...]
        r = jax.nn.sigmoid(gi[0:16] + gh[0:16])
        z = jax.nn.sigmoid(gi[16:32] + gh[16:32])
        n = jnp.tanh(gi[32:48] + r * gh[32:48])
        hnew_ref[...] = (1.0 - z) * n + z * h


def kernel(question, score, questions, hs, Ws, bs, W_ih, W_hh, b_ih, b_hh):
    f32 = jnp.float32
    q2 = question.reshape(QUES, 1)
    h_prev = hs[T - 1, 0]
    h2 = h_prev.reshape(SEQH, 1)
    sel = (score[0] < 0.5).astype(jnp.int32).reshape(1)

    alpha = pl.pallas_call(
        _alpha_body,
        grid=(GA,),
        in_specs=[
            pl.BlockSpec((QUES, 1), lambda i: (0, 0)),
            pl.BlockSpec((AROWS, QUES), lambda i: (i, 0)),
        ],
        out_specs=pl.BlockSpec((AROWS, 1), lambda i: (i, 0)),
        out_shape=jax.ShapeDtypeStruct((T, 1), f32),
    )(q2, questions)

    idx, vals = _sc_topk(alpha.reshape(T))

    grid_spec = pltpu.PrefetchScalarGridSpec(
        num_scalar_prefetch=1,
        grid=(GW,),
        in_specs=[
            pl.BlockSpec((QUES, 1), lambda i, s: (0, 0)),
            pl.BlockSpec((SEQH, 1), lambda i, s: (0, 0)),
            pl.BlockSpec((WROWS, QUES), lambda i, s: (i, s[0])),
            pl.BlockSpec((WROWS, SEQH), lambda i, s: (i, 0)),
        ],
        out_specs=[
            pl.BlockSpec((WROWS, 1), lambda i, s: (i, 0)),
            pl.BlockSpec((WROWS, 1), lambda i, s: (i, 0)),
        ],
    )
    gi, gh = pl.pallas_call(
        _matvec_body,
        grid_spec=grid_spec,
        out_shape=[
            jax.ShapeDtypeStruct((3 * SEQH, 1), f32),
            jax.ShapeDtypeStruct((3 * SEQH, 1), f32),
        ],
    )(sel, q2, h2, W_ih, W_hh)

    pred, h_new = pl.pallas_call(
        _final_body,
        grid_spec=pltpu.PrefetchScalarGridSpec(
            num_scalar_prefetch=1,
            grid=(K,),
            in_specs=[
                pl.BlockSpec((1, K), lambda i, s: (0, 0)),
                pl.BlockSpec((1, 1, SEQH), lambda i, s: (s[i], 0, 0)),
                pl.BlockSpec((1, QUES), lambda i, s: (0, 0)),
                pl.BlockSpec((2, QUES), lambda i, s: (0, 0)),
                pl.BlockSpec((1, 1), lambda i, s: (0, 0)),
                pl.BlockSpec((48, 128), lambda i, s: (0, 0)),
                pl.BlockSpec((48, 128), lambda i, s: (0, 0)),
                pl.BlockSpec((16, 128), lambda i, s: (0, 0)),
                pl.BlockSpec((48, 128), lambda i, s: (0, 0)),
                pl.BlockSpec((48, 128), lambda i, s: (0, 0)),
            ],
            out_specs=[
                pl.BlockSpec((1, 1), lambda i, s: (0, 0)),
                pl.BlockSpec((16, 128), lambda i, s: (0, 0)),
            ],
            scratch_shapes=[
                pltpu.VMEM((1, SEQH), f32),
                pltpu.VMEM((1, K), f32),
            ],
        ),
        out_shape=[
            jax.ShapeDtypeStruct((1, 1), f32),
            jax.ShapeDtypeStruct((16, 128), f32),
        ],
    )(
        idx, vals.reshape(1, K), hs,
        question.reshape(1, QUES), Ws.reshape(2, QUES), bs.reshape(1, 1),
        gi.reshape(48, 128), gh.reshape(48, 128), h_prev.reshape(16, 128),
        b_ih.reshape(48, 128), b_hh.reshape(48, 128),
    )
    return (pred.reshape(1), h_new.reshape(1, 1, SEQH))
